# baseline (speedup 1.0000x reference)
import functools

import jax
import jax.numpy as jnp
from jax import lax
from jax.experimental import pallas as pl
from jax.experimental.pallas import tpu as pltpu
from jax.experimental.pallas import tpu_sc as plsc

N = 10000
E = 640000
D_IN = 1433
H = 128
L = 8
C = 7

NC = 2
NS = 16
NW = NC * NS
HW = H // NC
LANE = 256
G = 10
SB = E // (G * LANE)
RPT = 640
NP = NS * RPT


def _sc_degree(dst3):
  mesh = plsc.VectorSubcoreMesh(core_axis_name="c", subcore_axis_name="s")

  @functools.partial(
      pl.kernel, mesh=mesh,
      out_type=jax.ShapeDtypeStruct((NC, NP, 16), jnp.float32),
      compiler_params=pltpu.CompilerParams(use_tc_tiling_on_sc=False),
      scratch_types=[
          pltpu.VMEM((G, LANE), jnp.int32),
          pltpu.VMEM((LANE, 16), jnp.float32),
          pltpu.VMEM((RPT, 16), jnp.float32),
          pltpu.VMEM_SHARED((NP, 16), jnp.float32),
      ],
  )
  def k(dst_hbm, out_hbm, didx_v, ones_v, zer_v, acc_sh):
    c = lax.axis_index("c")
    s = lax.axis_index("s")
    w = c * NS + s

    def fill(i, _):
      ones_v[i, :] = jnp.ones((16,), jnp.float32)
      return 0

    lax.fori_loop(0, LANE, fill, 0)

    def zfill(i, _):
      zer_v[i, :] = jnp.zeros((16,), jnp.float32)
      return 0

    lax.fori_loop(0, RPT, zfill, 0)
    pltpu.sync_copy(zer_v, acc_sh.at[pl.ds(s * RPT, RPT)])
    plsc.subcore_barrier()

    nt = (SB - w + NW - 1) // NW

    def body(t, _):
      sb = t * NW + w
      pltpu.sync_copy(dst_hbm.at[sb], didx_v)
      for j in range(G):
        pltpu.sync_copy(ones_v, acc_sh.at[didx_v.at[j]], add=True)
      return 0

    lax.fori_loop(0, nt, body, 0)
    plsc.subcore_barrier()
    pltpu.sync_copy(acc_sh.at[pl.ds(s * RPT, RPT)], zer_v)
    pltpu.sync_copy(zer_v, out_hbm.at[c, pl.ds(s * RPT, RPT)])

  return k(dst3)


def _sc_msgpass(g2, srcs, dst3):
  mesh = plsc.VectorSubcoreMesh(core_axis_name="c", subcore_axis_name="s")

  @functools.partial(
      pl.kernel, mesh=mesh,
      out_type=jax.ShapeDtypeStruct((NC, NP, HW), jnp.float32),
      compiler_params=pltpu.CompilerParams(use_tc_tiling_on_sc=False),
      scratch_types=[
          pltpu.VMEM((2, G, LANE), jnp.int32),
          pltpu.VMEM((2, G, LANE), jnp.int32),
          pltpu.VMEM((LANE, HW), jnp.float32),
          pltpu.VMEM((LANE, HW), jnp.float32),
          pltpu.VMEM((LANE, HW), jnp.float32),
          pltpu.VMEM((LANE, HW), jnp.float32),
          pltpu.VMEM_SHARED((NP, HW), jnp.float32),
          pltpu.SemaphoreType.DMA,
          pltpu.SemaphoreType.DMA,
          pltpu.SemaphoreType.DMA,
          pltpu.SemaphoreType.DMA,
          pltpu.SemaphoreType.DMA,
          pltpu.SemaphoreType.DMA,
          pltpu.SemaphoreType.DMA,
          pltpu.SemaphoreType.DMA,
          pltpu.SemaphoreType.DMA,
          pltpu.SemaphoreType.DMA,
      ],
  )
  def k(g_hbm, src_hbm, dst_hbm, out_hbm, sidx_v, didx_v, rows0_v, rows1_v,
        rows2_v, rows3_v, acc_sh, gsem0, gsem1, gsem2, gsem3,
        ssem0, ssem1, ssem2, ssem3, isem_s, isem_d):
    c = lax.axis_index("c")
    s = lax.axis_index("s")
    rows = (rows0_v, rows1_v, rows2_v, rows3_v)
    gsem = (gsem0, gsem1, gsem2, gsem3)
    ssem = (ssem0, ssem1, ssem2, ssem3)

    def zfill(i, _):
      for j in range(HW // 16):
        rows0_v[i, pl.ds(j * 16, 16)] = jnp.zeros((16,), jnp.float32)
      return 0

    lax.fori_loop(0, LANE, zfill, 0)
    off = 0
    while off < RPT:
      step = min(LANE, RPT - off)
      pltpu.sync_copy(rows0_v.at[pl.ds(0, step)],
                      acc_sh.at[pl.ds(s * RPT + off, step)])
      off += step
    plsc.subcore_barrier()

    nt = (SB - s + NS - 1) // NS

    pltpu.async_copy(src_hbm.at[c, s], sidx_v.at[0], isem_s)
    pltpu.async_copy(dst_hbm.at[s], didx_v.at[0], isem_d)

    def body(t, _):
      p = lax.rem(t, 2)
      sb = t * NS + s
      pltpu.make_async_copy(src_hbm.at[c, 0], sidx_v.at[p], isem_s).wait()
      pltpu.make_async_copy(dst_hbm.at[0], didx_v.at[p], isem_d).wait()
      nsb = jnp.minimum(sb + NS, SB - 1)
      pltpu.async_copy(src_hbm.at[c, nsb], sidx_v.at[1 - p], isem_s)
      pltpu.async_copy(dst_hbm.at[nsb], didx_v.at[1 - p], isem_d)
      dg = [None] * G
      dsc = [None] * G
      for j in range(min(3, G)):
        dg[j] = pltpu.async_copy(
            g_hbm.at[sidx_v.at[p, j]], rows[j % 4], gsem[j % 4])
      for j in range(G):
        b = j % 4
        if j + 3 < G:
          if j >= 1:
            dsc[j - 1].wait()
          dg[j + 3] = pltpu.async_copy(
              g_hbm.at[sidx_v.at[p, j + 3]], rows[(j + 3) % 4],
              gsem[(j + 3) % 4])
        dg[j].wait()
        dsc[j] = pltpu.async_copy(
            rows[b], acc_sh.at[didx_v.at[p, j]], ssem[b], add=True)
      for j in range(max(G - 4, 0), G):
        dsc[j].wait()
      return 0

    lax.fori_loop(0, nt, body, 0)
    pltpu.make_async_copy(src_hbm.at[c, 0], sidx_v.at[0], isem_s).wait()
    pltpu.make_async_copy(dst_hbm.at[0], didx_v.at[0], isem_d).wait()
    plsc.subcore_barrier()
    off = 0
    while off < RPT:
      step = min(LANE, RPT - off)
      pltpu.sync_copy(acc_sh.at[pl.ds(s * RPT + off, step)],
                      rows0_v.at[pl.ds(0, step)])
      pltpu.sync_copy(rows0_v.at[pl.ds(0, step)],
                      out_hbm.at[c, pl.ds(s * RPT + off, step)])
      off += step

  return k(g2, srcs, dst3)


_BN = 2000


def _dot(a, b):
  return jnp.dot(a, b, preferred_element_type=jnp.float32,
                 precision=lax.Precision.HIGHEST)


def _split_g(g_ref, gn):
  g_ref[0, :, :] = gn[:, :HW]
  g_ref[1, :, :] = gn[:, HW:]


def _tc_layer0(degp, x, W0):
  def body(degp_ref, x_ref, w_ref, dis_ref, g_ref):
    deg = degp_ref[0] + degp_ref[1] + 1.0
    dis = lax.rsqrt(deg)
    dis_ref[...] = dis
    _split_g(g_ref, dis[:, 0:1] * _dot(x_ref[...], w_ref[...]))

  return pl.pallas_call(
      body,
      grid=(N // _BN,),
      in_specs=[
          pl.BlockSpec((NC, _BN, 16), lambda i: (0, i, 0)),
          pl.BlockSpec((_BN, D_IN), lambda i: (i, 0)),
          pl.BlockSpec((D_IN, H), lambda i: (0, 0)),
      ],
      out_specs=[
          pl.BlockSpec((_BN, 16), lambda i: (i, 0)),
          pl.BlockSpec((NC, _BN, HW), lambda i: (0, i, 0)),
      ],
      out_shape=[
          jax.ShapeDtypeStruct((N, 16), jnp.float32),
          jax.ShapeDtypeStruct((NC, N, HW), jnp.float32),
      ],
  )(degp, x, W0)


def _tc_mid(P, g, dis16, b, Wn):
  def body(p_ref, g_ref, dis_ref, b_ref, w_ref, h_ref, gn_ref):
    dis = dis_ref[:, 0:1]
    ps = jnp.concatenate([p_ref[0] + g_ref[0], p_ref[1] + g_ref[1]], axis=1)
    h = jnp.maximum(dis * ps + b_ref[...], 0.0)
    h_ref[...] = h
    _split_g(gn_ref, dis * _dot(h, w_ref[...]))

  return pl.pallas_call(
      body,
      grid=(N // _BN,),
      in_specs=[
          pl.BlockSpec((NC, _BN, HW), lambda i: (0, i, 0)),
          pl.BlockSpec((NC, _BN, HW), lambda i: (0, i, 0)),
          pl.BlockSpec((_BN, 16), lambda i: (i, 0)),
          pl.BlockSpec((1, H), lambda i: (0, 0)),
          pl.BlockSpec((H, H), lambda i: (0, 0)),
      ],
      out_specs=[
          pl.BlockSpec((_BN, H), lambda i: (i, 0)),
          pl.BlockSpec((NC, _BN, HW), lambda i: (0, i, 0)),
      ],
      out_shape=[
          jax.ShapeDtypeStruct((N, H), jnp.float32),
          jax.ShapeDtypeStruct((NC, N, HW), jnp.float32),
      ],
  )(P, g, dis16, b, Wn)


def _tc_final(P, g, dis16, b, hs, linW3, linb):
  def body(p_ref, g_ref, dis_ref, b_ref, hs_ref, lw_ref, lb_ref, out_ref):
    dis = dis_ref[:, 0:1]
    ps = jnp.concatenate([p_ref[0] + g_ref[0], p_ref[1] + g_ref[1]], axis=1)
    h8 = jnp.maximum(dis * ps + b_ref[...], 0.0)
    z = lb_ref[...] + _dot(h8, lw_ref[L - 1])
    for l in range(L - 1):
      z = z + _dot(hs_ref[l], lw_ref[l])
    m = jnp.max(z, axis=1, keepdims=True)
    lse = m + jnp.log(jnp.sum(jnp.exp(z - m), axis=1, keepdims=True))
    out_ref[...] = z - lse

  return pl.pallas_call(
      body,
      grid=(N // _BN,),
      in_specs=[
          pl.BlockSpec((NC, _BN, HW), lambda i: (0, i, 0)),
          pl.BlockSpec((NC, _BN, HW), lambda i: (0, i, 0)),
          pl.BlockSpec((_BN, 16), lambda i: (i, 0)),
          pl.BlockSpec((1, H), lambda i: (0, 0)),
          pl.BlockSpec((L - 1, _BN, H), lambda i: (0, i, 0)),
          pl.BlockSpec((L, H, C), lambda i: (0, 0, 0)),
          pl.BlockSpec((1, C), lambda i: (0, 0)),
      ],
      out_specs=pl.BlockSpec((_BN, C), lambda i: (i, 0)),
      out_shape=jax.ShapeDtypeStruct((N, C), jnp.float32),
  )(P, g, dis16, b, hs, linW3, linb)


def kernel(x, edge_index, W0, b0, Ws, bs, lin_W, lin_b):
  src3 = edge_index[0].reshape(SB, G, LANE)
  dst3 = edge_index[1].reshape(SB, G, LANE)
  srcs = jnp.stack([src3, src3 + N])

  degp = _sc_degree(dst3)
  dis16, g = _tc_layer0(degp, x, W0)

  hs = []
  for l in range(L - 1):
    P = _sc_msgpass(g.reshape(NC * N, HW), srcs, dst3)
    b = (b0 if l == 0 else bs[l - 1]).reshape(1, H)
    h, g = _tc_mid(P, g, dis16, b, Ws[l])
    hs.append(h)

  P = _sc_msgpass(g.reshape(NC * N, HW), srcs, dst3)
  out = _tc_final(P, g, dis16, bs[L - 2].reshape(1, H),
                  jnp.stack(hs), lin_W.reshape(L, H, C), lin_b.reshape(1, C))
  return out


# speedup vs baseline: 1.0246x; 1.0246x over previous
import functools

import jax
import jax.numpy as jnp
from jax import lax
from jax.experimental import pallas as pl
from jax.experimental.pallas import tpu as pltpu
from jax.experimental.pallas import tpu_sc as plsc

N = 10000
E = 640000
D_IN = 1433
H = 128
L = 8
C = 7

NC = 2
NS = 16
NW = NC * NS
HW = H // NC
LANE = 256
G = 10
SB = E // (G * LANE)
RPT = 640
NP = NS * RPT


def _sc_degree(dst3):
  mesh = plsc.VectorSubcoreMesh(core_axis_name="c", subcore_axis_name="s")

  @functools.partial(
      pl.kernel, mesh=mesh,
      out_type=jax.ShapeDtypeStruct((NC, NP, 16), jnp.float32),
      compiler_params=pltpu.CompilerParams(use_tc_tiling_on_sc=False),
      scratch_types=[
          pltpu.VMEM((G, LANE), jnp.int32),
          pltpu.VMEM((LANE, 16), jnp.float32),
          pltpu.VMEM((RPT, 16), jnp.float32),
          pltpu.VMEM_SHARED((NP, 16), jnp.float32),
      ],
  )
  def k(dst_hbm, out_hbm, didx_v, ones_v, zer_v, acc_sh):
    c = lax.axis_index("c")
    s = lax.axis_index("s")
    w = c * NS + s

    def fill(i, _):
      ones_v[i, :] = jnp.ones((16,), jnp.float32)
      return 0

    lax.fori_loop(0, LANE, fill, 0)

    def zfill(i, _):
      zer_v[i, :] = jnp.zeros((16,), jnp.float32)
      return 0

    lax.fori_loop(0, RPT, zfill, 0)
    pltpu.sync_copy(zer_v, acc_sh.at[pl.ds(s * RPT, RPT)])
    plsc.subcore_barrier()

    nt = (SB - w + NW - 1) // NW

    def body(t, _):
      sb = t * NW + w
      pltpu.sync_copy(dst_hbm.at[sb], didx_v)
      for j in range(G):
        pltpu.sync_copy(ones_v, acc_sh.at[didx_v.at[j]], add=True)
      return 0

    lax.fori_loop(0, nt, body, 0)
    plsc.subcore_barrier()
    pltpu.sync_copy(acc_sh.at[pl.ds(s * RPT, RPT)], zer_v)
    pltpu.sync_copy(zer_v, out_hbm.at[c, pl.ds(s * RPT, RPT)])

  return k(dst3)


def _sc_msgpass(g2, srcs, dst3):
  mesh = plsc.VectorSubcoreMesh(core_axis_name="c", subcore_axis_name="s")

  @functools.partial(
      pl.kernel, mesh=mesh,
      out_type=jax.ShapeDtypeStruct((NC, NP, HW), jnp.float32),
      compiler_params=pltpu.CompilerParams(use_tc_tiling_on_sc=False),
      scratch_types=[
          pltpu.VMEM((2, G, LANE), jnp.int32),
          pltpu.VMEM((2, G, LANE), jnp.int32),
          pltpu.VMEM((LANE, HW), jnp.float32),
          pltpu.VMEM((LANE, HW), jnp.float32),
          pltpu.VMEM((LANE, HW), jnp.float32),
          pltpu.VMEM_SHARED((NP, HW), jnp.float32),
          pltpu.SemaphoreType.DMA,
          pltpu.SemaphoreType.DMA,
          pltpu.SemaphoreType.DMA,
          pltpu.SemaphoreType.DMA,
          pltpu.SemaphoreType.DMA,
          pltpu.SemaphoreType.DMA,
          pltpu.SemaphoreType.DMA,
          pltpu.SemaphoreType.DMA,
      ],
  )
  def k(g_hbm, src_hbm, dst_hbm, out_hbm, sidx_v, didx_v, rows0_v, rows1_v,
        rows2_v, acc_sh, gsem0, gsem1, gsem2, ssem0, ssem1, ssem2,
        isem_s, isem_d):
    c = lax.axis_index("c")
    s = lax.axis_index("s")
    rows = (rows0_v, rows1_v, rows2_v)
    gsem = (gsem0, gsem1, gsem2)
    ssem = (ssem0, ssem1, ssem2)

    def zfill(i, _):
      for j in range(HW // 16):
        rows0_v[i, pl.ds(j * 16, 16)] = jnp.zeros((16,), jnp.float32)
      return 0

    lax.fori_loop(0, LANE, zfill, 0)
    off = 0
    while off < RPT:
      step = min(LANE, RPT - off)
      pltpu.sync_copy(rows0_v.at[pl.ds(0, step)],
                      acc_sh.at[pl.ds(s * RPT + off, step)])
      off += step
    plsc.subcore_barrier()

    nt = (SB - s + NS - 1) // NS

    pltpu.async_copy(src_hbm.at[c, s], sidx_v.at[0], isem_s)
    pltpu.async_copy(dst_hbm.at[s], didx_v.at[0], isem_d)

    def body(t, _):
      p = lax.rem(t, 2)
      sb = t * NS + s
      pltpu.make_async_copy(src_hbm.at[c, 0], sidx_v.at[p], isem_s).wait()
      pltpu.make_async_copy(dst_hbm.at[0], didx_v.at[p], isem_d).wait()
      nsb = jnp.minimum(sb + NS, SB - 1)
      pltpu.async_copy(src_hbm.at[c, nsb], sidx_v.at[1 - p], isem_s)
      pltpu.async_copy(dst_hbm.at[nsb], didx_v.at[1 - p], isem_d)
      dg = [None] * G
      dsc = [None] * G
      for j in range(min(2, G)):
        dg[j] = pltpu.async_copy(
            g_hbm.at[sidx_v.at[p, j]], rows[j % 3], gsem[j % 3])
      for j in range(G):
        b = j % 3
        if j + 2 < G:
          if j >= 1:
            dsc[j - 1].wait()
          dg[j + 2] = pltpu.async_copy(
              g_hbm.at[sidx_v.at[p, j + 2]], rows[(j + 2) % 3],
              gsem[(j + 2) % 3])
        dg[j].wait()
        dsc[j] = pltpu.async_copy(
            rows[b], acc_sh.at[didx_v.at[p, j]], ssem[b], add=True)
      for j in range(max(G - 3, 0), G):
        dsc[j].wait()
      return 0

    lax.fori_loop(0, nt, body, 0)
    pltpu.make_async_copy(src_hbm.at[c, 0], sidx_v.at[0], isem_s).wait()
    pltpu.make_async_copy(dst_hbm.at[0], didx_v.at[0], isem_d).wait()
    plsc.subcore_barrier()
    off = 0
    while off < RPT:
      step = min(LANE, RPT - off)
      pltpu.sync_copy(acc_sh.at[pl.ds(s * RPT + off, step)],
                      rows0_v.at[pl.ds(0, step)])
      pltpu.sync_copy(rows0_v.at[pl.ds(0, step)],
                      out_hbm.at[c, pl.ds(s * RPT + off, step)])
      off += step

  return k(g2, srcs, dst3)


_BN = 2000


def _dot(a, b):
  return jnp.dot(a, b, preferred_element_type=jnp.float32,
                 precision=lax.Precision.HIGHEST)


def _split_g(g_ref, gn):
  g_ref[0, :, :] = gn[:, :HW]
  g_ref[1, :, :] = gn[:, HW:]


def _tc_layer0(degp, x, W0):
  def body(degp_ref, x_ref, w_ref, dis_ref, g_ref):
    deg = degp_ref[0] + degp_ref[1] + 1.0
    dis = lax.rsqrt(deg)
    dis_ref[...] = dis
    _split_g(g_ref, dis[:, 0:1] * _dot(x_ref[...], w_ref[...]))

  return pl.pallas_call(
      body,
      grid=(N // _BN,),
      in_specs=[
          pl.BlockSpec((NC, _BN, 16), lambda i: (0, i, 0)),
          pl.BlockSpec((_BN, D_IN), lambda i: (i, 0)),
          pl.BlockSpec((D_IN, H), lambda i: (0, 0)),
      ],
      out_specs=[
          pl.BlockSpec((_BN, 16), lambda i: (i, 0)),
          pl.BlockSpec((NC, _BN, HW), lambda i: (0, i, 0)),
      ],
      out_shape=[
          jax.ShapeDtypeStruct((N, 16), jnp.float32),
          jax.ShapeDtypeStruct((NC, N, HW), jnp.float32),
      ],
  )(degp, x, W0)


def _tc_mid(P, g, dis16, b, Wn):
  def body(p_ref, g_ref, dis_ref, b_ref, w_ref, h_ref, gn_ref):
    dis = dis_ref[:, 0:1]
    ps = jnp.concatenate([p_ref[0] + g_ref[0], p_ref[1] + g_ref[1]], axis=1)
    h = jnp.maximum(dis * ps + b_ref[...], 0.0)
    h_ref[...] = h
    _split_g(gn_ref, dis * _dot(h, w_ref[...]))

  return pl.pallas_call(
      body,
      grid=(N // _BN,),
      in_specs=[
          pl.BlockSpec((NC, _BN, HW), lambda i: (0, i, 0)),
          pl.BlockSpec((NC, _BN, HW), lambda i: (0, i, 0)),
          pl.BlockSpec((_BN, 16), lambda i: (i, 0)),
          pl.BlockSpec((1, H), lambda i: (0, 0)),
          pl.BlockSpec((H, H), lambda i: (0, 0)),
      ],
      out_specs=[
          pl.BlockSpec((_BN, H), lambda i: (i, 0)),
          pl.BlockSpec((NC, _BN, HW), lambda i: (0, i, 0)),
      ],
      out_shape=[
          jax.ShapeDtypeStruct((N, H), jnp.float32),
          jax.ShapeDtypeStruct((NC, N, HW), jnp.float32),
      ],
  )(P, g, dis16, b, Wn)


def _tc_final(P, g, dis16, b, hs, linW3, linb):
  def body(p_ref, g_ref, dis_ref, b_ref, hs_ref, lw_ref, lb_ref, out_ref):
    dis = dis_ref[:, 0:1]
    ps = jnp.concatenate([p_ref[0] + g_ref[0], p_ref[1] + g_ref[1]], axis=1)
    h8 = jnp.maximum(dis * ps + b_ref[...], 0.0)
    z = lb_ref[...] + _dot(h8, lw_ref[L - 1])
    for l in range(L - 1):
      z = z + _dot(hs_ref[l], lw_ref[l])
    m = jnp.max(z, axis=1, keepdims=True)
    lse = m + jnp.log(jnp.sum(jnp.exp(z - m), axis=1, keepdims=True))
    out_ref[...] = z - lse

  return pl.pallas_call(
      body,
      grid=(N // _BN,),
      in_specs=[
          pl.BlockSpec((NC, _BN, HW), lambda i: (0, i, 0)),
          pl.BlockSpec((NC, _BN, HW), lambda i: (0, i, 0)),
          pl.BlockSpec((_BN, 16), lambda i: (i, 0)),
          pl.BlockSpec((1, H), lambda i: (0, 0)),
          pl.BlockSpec((L - 1, _BN, H), lambda i: (0, i, 0)),
          pl.BlockSpec((L, H, C), lambda i: (0, 0, 0)),
          pl.BlockSpec((1, C), lambda i: (0, 0)),
      ],
      out_specs=pl.BlockSpec((_BN, C), lambda i: (i, 0)),
      out_shape=jax.ShapeDtypeStruct((N, C), jnp.float32),
  )(P, g, dis16, b, hs, linW3, linb)


def kernel(x, edge_index, W0, b0, Ws, bs, lin_W, lin_b):
  src3 = edge_index[0].reshape(SB, G, LANE)
  dst3 = edge_index[1].reshape(SB, G, LANE)
  srcs = jnp.stack([src3, src3 + N])

  degp = _sc_degree(dst3)
  dis16, g = _tc_layer0(degp, x, W0)

  hs = []
  for l in range(L - 1):
    P = _sc_msgpass(g.reshape(NC * N, HW), srcs, dst3)
    b = (b0 if l == 0 else bs[l - 1]).reshape(1, H)
    h, g = _tc_mid(P, g, dis16, b, Ws[l])
    hs.append(h)

  P = _sc_msgpass(g.reshape(NC * N, HW), srcs, dst3)
  out = _tc_final(P, g, dis16, bs[L - 2].reshape(1, H),
                  jnp.stack(hs), lin_W.reshape(L, H, C), lin_b.reshape(1, C))
  return out
